# baseline (speedup 1.0000x reference)
import jax
import jax.numpy as jnp
from jax import lax
from jax.experimental import pallas as pl
from jax.experimental.pallas import tpu as pltpu
from jax.experimental.pallas import tpu_sc as plsc

D_MODEL = 64
SCALE = 8.0
PAD = 0
LANES = 16

NUM_CORES = 2
NUM_SUBCORES = 16
NW = NUM_CORES * NUM_SUBCORES

BATCH = 1024 * 200
ROWS_PER_W = BATCH // NW
GROUP = 128
NGROUPS = ROWS_PER_W // GROUP

_GATHER_DNUMS = lax.GatherDimensionNumbers(
    offset_dims=(), collapsed_slice_dims=(0,), start_index_map=(0,))


def _emb_body(idx_hbm, table_hbm, out_hbm, idx_v, buf, sem):
    w = lax.axis_index("s") * NUM_CORES + lax.axis_index("c")

    pltpu.sync_copy(idx_hbm.at[w], idx_v)

    def group_body(g, carry):
        pltpu.async_copy(table_hbm.at[idx_v.at[g]], buf, sem).wait()

        def j_body(j, c2):
            idx16 = idx_v[g, pl.ds(j * LANES, LANES)]
            s16 = jnp.where(idx16 == PAD, 0.0, SCALE).astype(jnp.float32)
            for r in range(LANES):
                sv = lax.gather(
                    s16, jnp.full((LANES, 1), r, jnp.int32), _GATHER_DNUMS,
                    (1,), mode=lax.GatherScatterMode.PROMISE_IN_BOUNDS)
                row = j * LANES + r
                for c in range(D_MODEL // LANES):
                    sl = pl.ds(c * LANES, LANES)
                    buf[row, sl] = buf[row, sl] * sv
            return c2

        lax.fori_loop(0, GROUP // LANES, j_body, 0)

        pltpu.sync_copy(buf, out_hbm.at[pl.ds(w * ROWS_PER_W + g * GROUP, GROUP)])
        return carry

    lax.fori_loop(0, NGROUPS, group_body, 0)


def kernel(x, table):
    idx3 = x.reshape(NW, NGROUPS, GROUP).astype(jnp.int32)
    mesh = plsc.VectorSubcoreMesh(core_axis_name="c", subcore_axis_name="s")
    out = pl.kernel(
        _emb_body,
        mesh=mesh,
        compiler_params=pltpu.CompilerParams(use_tc_tiling_on_sc=False),
        out_type=jax.ShapeDtypeStruct((BATCH, D_MODEL), jnp.float32),
        scratch_types=[
            pltpu.VMEM((NGROUPS, GROUP), jnp.int32),
            pltpu.VMEM((GROUP, D_MODEL), jnp.float32),
            pltpu.SemaphoreType.DMA,
        ],
    )(idx3, table)
    return out.reshape(x.shape[0], x.shape[1], D_MODEL)


# speedup vs baseline: 1.7285x; 1.7285x over previous
import jax
import jax.numpy as jnp
from jax import lax
from jax.experimental import pallas as pl
from jax.experimental.pallas import tpu as pltpu
from jax.experimental.pallas import tpu_sc as plsc

D_MODEL = 64
LANES = 16
NUM_CORES = 2
NW = 32
BATCH = 1024 * 200
ROWS_PER_W = BATCH // NW
GROUP = 128
NGROUPS = ROWS_PER_W // GROUP
SCALE = 8.0
PAD = 0


def _emb_body(idx_hbm, table_hbm, out_hbm, idx_v, buf, sem):
    w = lax.axis_index("s") * NUM_CORES + lax.axis_index("c")
    pltpu.sync_copy(idx_hbm.at[pl.ds(w * ROWS_PER_W, ROWS_PER_W)], idx_v)

    def group_body(g, carry):
        base = g * GROUP

        def fire_body(j, c2):
            idx16 = idx_v[pl.ds(base + j * LANES, LANES)]
            for r in range(LANES):
                v = idx16[r]
                pltpu.async_copy(table_hbm.at[v], buf.at[j * LANES + r], sem)
            return c2

        lax.fori_loop(0, GROUP // LANES, fire_body, 0)
        pltpu.make_async_copy(table_hbm.at[pl.ds(0, GROUP)], buf, sem).wait()

        def j_body(j, c2):
            idx16 = idx_v[pl.ds(base + j * LANES, LANES)]
            s16 = jnp.where(idx16 == PAD, 0.0, SCALE).astype(jnp.float32)
            for r in range(LANES):
                sv = lax.broadcast_in_dim(s16[r], (LANES,), ())
                row = j * LANES + r
                for c in range(D_MODEL // LANES):
                    sl = pl.ds(c * LANES, LANES)
                    buf[row, sl] = buf[row, sl] * sv
            return c2

        lax.fori_loop(0, GROUP // LANES, j_body, 0)

        pltpu.sync_copy(buf, out_hbm.at[pl.ds(w * ROWS_PER_W + base, GROUP)])
        return carry

    lax.fori_loop(0, NGROUPS, group_body, 0)


def kernel(x, table):
    idx1 = x.reshape(BATCH).astype(jnp.int32)
    mesh = plsc.VectorSubcoreMesh(core_axis_name="c", subcore_axis_name="s")
    out = pl.kernel(
        _emb_body,
        mesh=mesh,
        out_type=jax.ShapeDtypeStruct((BATCH, D_MODEL), jnp.float32),
        scratch_types=[
            pltpu.VMEM((ROWS_PER_W,), jnp.int32),
            pltpu.VMEM((GROUP, D_MODEL), jnp.float32),
            pltpu.SemaphoreType.DMA,
        ],
    )(idx1, table)
    return out.reshape(x.shape[0], x.shape[1], D_MODEL)
